# baseline (speedup 1.0000x reference)
import jax
import jax.numpy as jnp
from jax.experimental import pallas as pl

N = 10000; E = 320000; B = 64; L = 128; DESC = 80; DM = 32; NH = 4; FF = 128; DF = 128; OUT = 128; EPS = 1e-5


def _gcn_conv(x, edge_index, W, b):
    n = x.shape[0]
    h = x @ W.T
    src = edge_index[0]; dst = edge_index[1]
    loop = jnp.arange(n, dtype=src.dtype)
    src = jnp.concatenate([src, loop]); dst = jnp.concatenate([dst, loop])
    deg = jnp.zeros((n,), dtype=x.dtype).at[dst].add(1.0)
    dinv = jnp.where(deg > 0, deg ** -0.5, 0.0)
    norm = dinv[src] * dinv[dst]
    msg = h[src] * norm[:, None]
    out = jnp.zeros_like(h).at[dst].add(msg)
    return out + b


def _global_mean_pool(x, batch, num_graphs):
    s = jax.ops.segment_sum(x, batch, num_segments=num_graphs)
    c = jax.ops.segment_sum(jnp.ones((x.shape[0],), x.dtype), batch, num_segments=num_graphs)
    return s / jnp.clip(c, 1.0)[:, None]


def _layer_norm(x, g, b):
    m = x.mean(-1, keepdims=True)
    v = ((x - m) ** 2).mean(-1, keepdims=True)
    return (x - m) / jnp.sqrt(v + EPS) * g + b


def _mha(x, inW, inb, outW, outb):
    S, Bb, d = x.shape
    qkv = x @ inW.T + inb
    q, k, v = jnp.split(qkv, 3, axis=-1)
    dh = d // NH
    def rs(t):
        return t.reshape(S, Bb, NH, dh).transpose(1, 2, 0, 3)
    q = rs(q); k = rs(k); v = rs(v)
    a = jax.nn.softmax(jnp.einsum('bhsd,bhtd->bhst', q, k) / jnp.sqrt(dh), axis=-1)
    o = jnp.einsum('bhst,bhtd->bhsd', a, v)
    o = o.transpose(2, 0, 1, 3).reshape(S, Bb, d)
    return o @ outW.T + outb


def _encoder_layer(x, p, i):
    a = _mha(x, p['l%d_inW' % i], p['l%d_inb' % i], p['l%d_outW' % i], p['l%d_outb' % i])
    x = _layer_norm(x + a, p['l%d_ln1g' % i], p['l%d_ln1b' % i])
    f = jax.nn.relu(x @ p['l%d_ff1W' % i].T + p['l%d_ff1b' % i]) @ p['l%d_ff2W' % i].T + p['l%d_ff2b' % i]
    x = _layer_norm(x + f, p['l%d_ln2g' % i], p['l%d_ln2b' % i])
    return x


def _final_kernel(c_ref, w_ref, b_ref, o_ref):
    prod = c_ref[...] * w_ref[...]
    o_ref[...] = jnp.sum(prod, axis=1) + b_ref[0]


def kernel(pro1_x, pro1_edge_index, pro1_batch, pro2_x, pro2_edge_index, pro2_batch, mas1_straight, mas1_flipped, mas2_straight, mas2_flipped, params):
    p = params
    lrelu = lambda t: jax.nn.leaky_relu(t, 0.01)
    x = lrelu(_gcn_conv(pro1_x, pro1_edge_index, p['conv1_W'], p['conv1_b']))
    x = _global_mean_pool(x, pro1_batch, B)
    x = lrelu(x @ p['fc1_W'].T + p['fc1_b'])
    xt = lrelu(_gcn_conv(pro2_x, pro2_edge_index, p['conv2_W'], p['conv2_b']))
    xt = _global_mean_pool(xt, pro2_batch, B)
    xt = lrelu(xt @ p['fc2_W'].T + p['fc2_b'])
    ones = jnp.ones((B, L, 1), jnp.float32)
    zeros = jnp.zeros((B, L, 1), jnp.float32)
    red = lambda m: m @ p['red_W'].T + p['red_b']
    m1s = jnp.concatenate([red(mas1_straight), ones, ones], axis=-1)
    m1f = jnp.concatenate([red(mas1_flipped), zeros, ones], axis=-1)
    m2s = jnp.concatenate([red(mas2_straight), ones, zeros], axis=-1)
    m2f = jnp.concatenate([red(mas2_flipped), zeros, zeros], axis=-1)
    mas = jnp.concatenate([m1s, m1f, m2s, m2f], axis=1).transpose(1, 0, 2)
    for i in range(2):
        mas = _encoder_layer(mas, p, i)
    mas_out = mas.mean(axis=0)
    combined = jnp.concatenate([x, xt, mas_out], axis=1)
    out = pl.pallas_call(
        _final_kernel,
        out_shape=jax.ShapeDtypeStruct((B,), jnp.float32),
    )(combined, p['final_W'], p['final_b'])
    return out[:, None]


# speedup vs baseline: 6.3683x; 6.3683x over previous
import functools

import jax
import jax.numpy as jnp
from jax import lax
from jax.experimental import pallas as pl
from jax.experimental.pallas import tpu as pltpu
from jax.experimental.pallas import tpu_sc as plsc

N = 10000; E = 320000; B = 64; L = 128; DESC = 80; DM = 32; NH = 4; FF = 128; DF = 128; OUT = 128; EPS = 1e-5

NC = 2
NS = 16
CH = 128
NCHUNK = E // CH
NPS = 624
_NITER = (NCHUNK + NS - 1) // NS


def _sc_scatter_body(src1, dst1, src2, dst2, g1, g2, out1, out2,
                     sidx, didx, rows, sem_g, sem_s, acc):
    c = lax.axis_index("c")
    s = lax.axis_index("s")

    def per_graph(src, dst, g, out):
        nb = s * NPS
        pltpu.sync_copy(g.at[pl.ds(nb, NPS)], acc.at[pl.ds(nb, NPS)])
        @pl.when(s == NS - 1)
        def _():
            pltpu.sync_copy(g.at[pl.ds(NS * NPS, N - NS * NPS)],
                            acc.at[pl.ds(NS * NPS, N - NS * NPS)])
        plsc.subcore_barrier()

        def body(i, carry):
            k = i * NS + s
            @pl.when(k < NCHUNK)
            def _():
                base = k * CH
                pltpu.sync_copy(src.at[pl.ds(base, CH)], sidx)
                pltpu.sync_copy(dst.at[pl.ds(base, CH)], didx)
                pltpu.async_copy(g.at[sidx], rows, sem_g).wait()
                pltpu.async_copy(rows, acc.at[didx], sem_s, add=True).wait()
            return carry

        lax.fori_loop(0, _NITER, body, 0)
        plsc.subcore_barrier()
        pltpu.sync_copy(acc.at[pl.ds(nb, NPS)], out.at[pl.ds(nb, NPS)])
        @pl.when(s == NS - 1)
        def _():
            pltpu.sync_copy(acc.at[pl.ds(NS * NPS, N - NS * NPS)],
                            out.at[pl.ds(NS * NPS, N - NS * NPS)])

    @pl.when(c == 0)
    def _():
        per_graph(src1, dst1, g1, out1)

    @pl.when(c == 1)
    def _():
        per_graph(src2, dst2, g2, out2)


@jax.jit
def _sc_scatter(src1, dst1, src2, dst2, g1, g2):
    mesh = plsc.VectorSubcoreMesh(core_axis_name="c", subcore_axis_name="s",
                                  num_cores=NC, num_subcores=NS)
    return pl.kernel(
        _sc_scatter_body,
        out_type=[jax.ShapeDtypeStruct((N, DF), jnp.float32),
                  jax.ShapeDtypeStruct((N, DF), jnp.float32)],
        mesh=mesh,
        scratch_types=[
            pltpu.VMEM((CH,), jnp.int32),
            pltpu.VMEM((CH,), jnp.int32),
            pltpu.VMEM((CH, DF), jnp.float32),
            pltpu.SemaphoreType.DMA,
            pltpu.SemaphoreType.DMA,
            pltpu.VMEM_SHARED((N, DF), jnp.float32),
        ],
    )(src1, dst1, src2, dst2, g1, g2)


def _gcn_pair(x1, ei1, x2, ei2, p):
    h1 = x1 @ p['conv1_W'].T
    h2 = x2 @ p['conv2_W'].T
    deg1 = jnp.zeros((N,), jnp.float32).at[ei1[1]].add(1.0) + 1.0
    deg2 = jnp.zeros((N,), jnp.float32).at[ei2[1]].add(1.0) + 1.0
    dinv1 = deg1 ** -0.5
    dinv2 = deg2 ** -0.5
    g1 = h1 * dinv1[:, None]
    g2 = h2 * dinv2[:, None]
    s1, s2 = _sc_scatter(ei1[0], ei1[1], ei2[0], ei2[1], g1, g2)
    o1 = dinv1[:, None] * s1 + p['conv1_b']
    o2 = dinv2[:, None] * s2 + p['conv2_b']
    return o1, o2


def _global_mean_pool(x, batch, num_graphs):
    s = jax.ops.segment_sum(x, batch, num_segments=num_graphs)
    c = jax.ops.segment_sum(jnp.ones((x.shape[0],), x.dtype), batch, num_segments=num_graphs)
    return s / jnp.clip(c, 1.0)[:, None]


def _layer_norm(x, g, b):
    m = x.mean(-1, keepdims=True)
    v = ((x - m) ** 2).mean(-1, keepdims=True)
    return (x - m) / jnp.sqrt(v + EPS) * g + b


def _mha(x, inW, inb, outW, outb):
    S, Bb, d = x.shape
    qkv = x @ inW.T + inb
    q, k, v = jnp.split(qkv, 3, axis=-1)
    dh = d // NH
    def rs(t):
        return t.reshape(S, Bb, NH, dh).transpose(1, 2, 0, 3)
    q = rs(q); k = rs(k); v = rs(v)
    a = jax.nn.softmax(jnp.einsum('bhsd,bhtd->bhst', q, k) / jnp.sqrt(dh), axis=-1)
    o = jnp.einsum('bhst,bhtd->bhsd', a, v)
    o = o.transpose(2, 0, 1, 3).reshape(S, Bb, d)
    return o @ outW.T + outb


def _encoder_layer(x, p, i):
    a = _mha(x, p['l%d_inW' % i], p['l%d_inb' % i], p['l%d_outW' % i], p['l%d_outb' % i])
    x = _layer_norm(x + a, p['l%d_ln1g' % i], p['l%d_ln1b' % i])
    f = jax.nn.relu(x @ p['l%d_ff1W' % i].T + p['l%d_ff1b' % i]) @ p['l%d_ff2W' % i].T + p['l%d_ff2b' % i]
    x = _layer_norm(x + f, p['l%d_ln2g' % i], p['l%d_ln2b' % i])
    return x


def _final_kernel(c_ref, w_ref, b_ref, o_ref):
    prod = c_ref[...] * w_ref[...]
    o_ref[...] = jnp.sum(prod, axis=1) + b_ref[0]


def kernel(pro1_x, pro1_edge_index, pro1_batch, pro2_x, pro2_edge_index, pro2_batch, mas1_straight, mas1_flipped, mas2_straight, mas2_flipped, params):
    p = params
    lrelu = lambda t: jax.nn.leaky_relu(t, 0.01)
    conv1, conv2 = _gcn_pair(pro1_x, pro1_edge_index, pro2_x, pro2_edge_index, p)
    x = lrelu(conv1)
    x = _global_mean_pool(x, pro1_batch, B)
    x = lrelu(x @ p['fc1_W'].T + p['fc1_b'])
    xt = lrelu(conv2)
    xt = _global_mean_pool(xt, pro2_batch, B)
    xt = lrelu(xt @ p['fc2_W'].T + p['fc2_b'])
    ones = jnp.ones((B, L, 1), jnp.float32)
    zeros = jnp.zeros((B, L, 1), jnp.float32)
    red = lambda m: m @ p['red_W'].T + p['red_b']
    m1s = jnp.concatenate([red(mas1_straight), ones, ones], axis=-1)
    m1f = jnp.concatenate([red(mas1_flipped), zeros, ones], axis=-1)
    m2s = jnp.concatenate([red(mas2_straight), ones, zeros], axis=-1)
    m2f = jnp.concatenate([red(mas2_flipped), zeros, zeros], axis=-1)
    mas = jnp.concatenate([m1s, m1f, m2s, m2f], axis=1).transpose(1, 0, 2)
    for i in range(2):
        mas = _encoder_layer(mas, p, i)
    mas_out = mas.mean(axis=0)
    combined = jnp.concatenate([x, xt, mas_out], axis=1)
    out = pl.pallas_call(
        _final_kernel,
        out_shape=jax.ShapeDtypeStruct((B,), jnp.float32),
    )(combined, p['final_W'], p['final_b'])
    return out[:, None]


# speedup vs baseline: 7.7645x; 1.2193x over previous
import functools

import jax
import jax.numpy as jnp
from jax import lax
from jax.experimental import pallas as pl
from jax.experimental.pallas import tpu as pltpu
from jax.experimental.pallas import tpu_sc as plsc

N = 10000; E = 320000; B = 64; L = 128; DESC = 80; DM = 32; NH = 4; FF = 128; DF = 128; OUT = 128; EPS = 1e-5

NC = 2
NS = 16
CH = 64
NB = 4
N_PAD = N + 8
NCHUNK = 5120
E_PAD = NCHUNK * CH
NIT = NCHUNK // NS
NBLK = NIT // NB
ZR = 128
NPS = 624
DEGW = 16

_MESH = plsc.VectorSubcoreMesh(core_axis_name="c", subcore_axis_name="s",
                               num_cores=NC, num_subcores=NS)


def _node_chunk_copy(s, src_ref, dst_ref):
    nb = s * NPS
    pltpu.sync_copy(src_ref.at[pl.ds(nb, NPS)], dst_ref.at[pl.ds(nb, NPS)])
    @pl.when(s == NS - 1)
    def _():
        pltpu.sync_copy(src_ref.at[pl.ds(NS * NPS, N - NS * NPS)],
                        dst_ref.at[pl.ds(NS * NPS, N - NS * NPS)])


def _sc_deg_body(dst1, dst2, out1, out2, didx, ones, zeros, sem_s, acc):
    c = lax.axis_index("c")
    s = lax.axis_index("s")
    for r in range(CH):
        ones[r] = jnp.full((DEGW,), 1.0, jnp.float32)
    for r in range(ZR):
        zeros[r] = jnp.zeros((DEGW,), jnp.float32)

    def per_graph(dst, out):
        nb = s * NPS
        for t in range(4):
            pltpu.sync_copy(zeros, acc.at[pl.ds(nb + t * ZR, ZR)])
        pltpu.sync_copy(zeros.at[pl.ds(0, NPS - 4 * ZR)],
                        acc.at[pl.ds(nb + 4 * ZR, NPS - 4 * ZR)])
        @pl.when(s == NS - 1)
        def _():
            pltpu.sync_copy(zeros.at[pl.ds(0, N_PAD - NS * NPS)],
                            acc.at[pl.ds(NS * NPS, N_PAD - NS * NPS)])
        plsc.subcore_barrier()

        def blk(j, carry):
            for b in range(NB):
                k = (j * NB + b) * NS + s
                @pl.when(j > 0)
                def _():
                    pltpu.make_async_copy(ones, acc.at[didx.at[b]],
                                          sem_s.at[b]).wait()
                pltpu.sync_copy(dst.at[pl.ds(k * CH, CH)], didx.at[b])
                pltpu.async_copy(ones, acc.at[didx.at[b]], sem_s.at[b],
                                 add=True)
            return carry

        lax.fori_loop(0, NBLK, blk, 0)
        for b in range(NB):
            pltpu.make_async_copy(ones, acc.at[didx.at[b]], sem_s.at[b]).wait()
        plsc.subcore_barrier()
        _node_chunk_copy(s, acc, out)

    @pl.when(c == 0)
    def _():
        per_graph(dst1, out1)

    @pl.when(c == 1)
    def _():
        per_graph(dst2, out2)


@jax.jit
def _sc_deg(dst1, dst2):
    return pl.kernel(
        _sc_deg_body,
        out_type=[jax.ShapeDtypeStruct((N, DEGW), jnp.float32),
                  jax.ShapeDtypeStruct((N, DEGW), jnp.float32)],
        mesh=_MESH,
        scratch_types=[
            pltpu.VMEM((NB, CH), jnp.int32),
            pltpu.VMEM((CH, DEGW), jnp.float32),
            pltpu.VMEM((ZR, DEGW), jnp.float32),
            pltpu.SemaphoreType.DMA((NB,)),
            pltpu.VMEM_SHARED((N_PAD, DEGW), jnp.float32),
        ],
    )(dst1, dst2)


def _sc_scatter_body(src1, dst1, src2, dst2, g1, g2, out1, out2,
                     sidx, didx, rows, sem_g, sem_s, acc):
    c = lax.axis_index("c")
    s = lax.axis_index("s")

    def per_graph(src, dst, g, out):
        _node_chunk_copy(s, g, acc)
        @pl.when(s == NS - 1)
        def _():
            pltpu.sync_copy(g.at[pl.ds(N, N_PAD - N)], acc.at[pl.ds(N, N_PAD - N)])
        plsc.subcore_barrier()

        def stage1(j, b):
            k = (j * NB + b) * NS + s
            pltpu.sync_copy(src.at[pl.ds(k * CH, CH)], sidx.at[b])
            pltpu.sync_copy(dst.at[pl.ds(k * CH, CH)], didx.at[b])
            pltpu.async_copy(g.at[sidx.at[b]], rows.at[b], sem_g.at[b])

        def stage2(b2):
            pltpu.make_async_copy(g.at[sidx.at[b2]], rows.at[b2],
                                  sem_g.at[b2]).wait()
            pltpu.async_copy(rows.at[b2], acc.at[didx.at[b2]], sem_s.at[b2],
                             add=True)

        def blk(j, carry):
            for b in range(NB):
                @pl.when(j > 0)
                def _():
                    pltpu.make_async_copy(rows.at[b], acc.at[didx.at[b]],
                                          sem_s.at[b]).wait()
                stage1(j, b)
                b2 = (b + NB - 2) % NB
                if b >= 2:
                    stage2(b2)
                else:
                    @pl.when(j > 0)
                    def _():
                        stage2(b2)
            return carry

        lax.fori_loop(0, NBLK, blk, 0)
        for b2 in (2, 3):
            stage2(b2)
        for b in range(NB):
            pltpu.make_async_copy(rows.at[b], acc.at[didx.at[b]],
                                  sem_s.at[b]).wait()
        plsc.subcore_barrier()
        _node_chunk_copy(s, acc, out)

    @pl.when(c == 0)
    def _():
        per_graph(src1, dst1, g1, out1)

    @pl.when(c == 1)
    def _():
        per_graph(src2, dst2, g2, out2)


@jax.jit
def _sc_scatter(src1, dst1, src2, dst2, g1, g2):
    return pl.kernel(
        _sc_scatter_body,
        out_type=[jax.ShapeDtypeStruct((N, DF), jnp.float32),
                  jax.ShapeDtypeStruct((N, DF), jnp.float32)],
        mesh=_MESH,
        scratch_types=[
            pltpu.VMEM((NB, CH), jnp.int32),
            pltpu.VMEM((NB, CH), jnp.int32),
            pltpu.VMEM((NB, CH, DF), jnp.float32),
            pltpu.SemaphoreType.DMA((NB,)),
            pltpu.SemaphoreType.DMA((NB,)),
            pltpu.VMEM_SHARED((N_PAD, DF), jnp.float32),
        ],
    )(src1, dst1, src2, dst2, g1, g2)


def _pad_edges(ei):
    pad = jnp.full((E_PAD - E,), N, jnp.int32)
    return (jnp.concatenate([ei[0], pad]), jnp.concatenate([ei[1], pad]))


def _gcn_pair(x1, ei1, x2, ei2, p):
    src1, dst1 = _pad_edges(ei1)
    src2, dst2 = _pad_edges(ei2)
    h1 = x1 @ p['conv1_W'].T
    h2 = x2 @ p['conv2_W'].T
    d1, d2 = _sc_deg(dst1, dst2)
    dinv1 = (d1[:, 0] + 1.0) ** -0.5
    dinv2 = (d2[:, 0] + 1.0) ** -0.5
    zpad = jnp.zeros((N_PAD - N, DF), jnp.float32)
    g1 = jnp.concatenate([h1 * dinv1[:, None], zpad])
    g2 = jnp.concatenate([h2 * dinv2[:, None], zpad])
    s1, s2 = _sc_scatter(src1, dst1, src2, dst2, g1, g2)
    o1 = dinv1[:, None] * s1 + p['conv1_b']
    o2 = dinv2[:, None] * s2 + p['conv2_b']
    return o1, o2


def _global_mean_pool(x, batch, num_graphs):
    s = jax.ops.segment_sum(x, batch, num_segments=num_graphs)
    c = jax.ops.segment_sum(jnp.ones((x.shape[0],), x.dtype), batch, num_segments=num_graphs)
    return s / jnp.clip(c, 1.0)[:, None]


def _layer_norm(x, g, b):
    m = x.mean(-1, keepdims=True)
    v = ((x - m) ** 2).mean(-1, keepdims=True)
    return (x - m) / jnp.sqrt(v + EPS) * g + b


def _mha(x, inW, inb, outW, outb):
    S, Bb, d = x.shape
    qkv = x @ inW.T + inb
    q, k, v = jnp.split(qkv, 3, axis=-1)
    dh = d // NH
    def rs(t):
        return t.reshape(S, Bb, NH, dh).transpose(1, 2, 0, 3)
    q = rs(q); k = rs(k); v = rs(v)
    a = jax.nn.softmax(jnp.einsum('bhsd,bhtd->bhst', q, k) / jnp.sqrt(dh), axis=-1)
    o = jnp.einsum('bhst,bhtd->bhsd', a, v)
    o = o.transpose(2, 0, 1, 3).reshape(S, Bb, d)
    return o @ outW.T + outb


def _encoder_layer(x, p, i):
    a = _mha(x, p['l%d_inW' % i], p['l%d_inb' % i], p['l%d_outW' % i], p['l%d_outb' % i])
    x = _layer_norm(x + a, p['l%d_ln1g' % i], p['l%d_ln1b' % i])
    f = jax.nn.relu(x @ p['l%d_ff1W' % i].T + p['l%d_ff1b' % i]) @ p['l%d_ff2W' % i].T + p['l%d_ff2b' % i]
    x = _layer_norm(x + f, p['l%d_ln2g' % i], p['l%d_ln2b' % i])
    return x


def _final_kernel(c_ref, w_ref, b_ref, o_ref):
    prod = c_ref[...] * w_ref[...]
    o_ref[...] = jnp.sum(prod, axis=1) + b_ref[0]


def kernel(pro1_x, pro1_edge_index, pro1_batch, pro2_x, pro2_edge_index, pro2_batch, mas1_straight, mas1_flipped, mas2_straight, mas2_flipped, params):
    p = params
    lrelu = lambda t: jax.nn.leaky_relu(t, 0.01)
    conv1, conv2 = _gcn_pair(pro1_x, pro1_edge_index, pro2_x, pro2_edge_index, p)
    x = lrelu(conv1)
    x = _global_mean_pool(x, pro1_batch, B)
    x = lrelu(x @ p['fc1_W'].T + p['fc1_b'])
    xt = lrelu(conv2)
    xt = _global_mean_pool(xt, pro2_batch, B)
    xt = lrelu(xt @ p['fc2_W'].T + p['fc2_b'])
    ones = jnp.ones((B, L, 1), jnp.float32)
    zeros = jnp.zeros((B, L, 1), jnp.float32)
    red = lambda m: m @ p['red_W'].T + p['red_b']
    m1s = jnp.concatenate([red(mas1_straight), ones, ones], axis=-1)
    m1f = jnp.concatenate([red(mas1_flipped), zeros, ones], axis=-1)
    m2s = jnp.concatenate([red(mas2_straight), ones, zeros], axis=-1)
    m2f = jnp.concatenate([red(mas2_flipped), zeros, zeros], axis=-1)
    mas = jnp.concatenate([m1s, m1f, m2s, m2f], axis=1).transpose(1, 0, 2)
    for i in range(2):
        mas = _encoder_layer(mas, p, i)
    mas_out = mas.mean(axis=0)
    combined = jnp.concatenate([x, xt, mas_out], axis=1)
    out = pl.pallas_call(
        _final_kernel,
        out_shape=jax.ShapeDtypeStruct((B,), jnp.float32),
    )(combined, p['final_W'], p['final_b'])
    return out[:, None]


# speedup vs baseline: 10.5044x; 1.3529x over previous
import functools

import jax
import jax.numpy as jnp
from jax import lax
from jax.experimental import pallas as pl
from jax.experimental.pallas import tpu as pltpu
from jax.experimental.pallas import tpu_sc as plsc

N = 10000; E = 320000; B = 64; L = 128; DESC = 80; DM = 32; NH = 4; FF = 128; DF = 128; OUT = 128; EPS = 1e-5

NC = 2
NS = 16
CH = 64
NB = 4
N_PAD = N + 8
NCHUNK = 5120
E_PAD = NCHUNK * CH
NIT = NCHUNK // NS
NBLK = NIT // NB
ZR = 128
NPS = 624
DEGW = 16

def _sc_mesh():
    return plsc.VectorSubcoreMesh(core_axis_name="c", subcore_axis_name="s",
                                  num_cores=NC, num_subcores=NS)


def _node_chunk_copy(s, src_ref, dst_ref):
    nb = s * NPS
    pltpu.sync_copy(src_ref.at[pl.ds(nb, NPS)], dst_ref.at[pl.ds(nb, NPS)])
    @pl.when(s == NS - 1)
    def _():
        pltpu.sync_copy(src_ref.at[pl.ds(NS * NPS, N - NS * NPS)],
                        dst_ref.at[pl.ds(NS * NPS, N - NS * NPS)])


def _sc_deg_body(dst1, dst2, out1, out2, didx, ones, zeros, sem_s, acc):
    c = lax.axis_index("c")
    s = lax.axis_index("s")
    for r in range(CH):
        ones[r] = jnp.full((DEGW,), 1.0, jnp.float32)
    for r in range(ZR):
        zeros[r] = jnp.zeros((DEGW,), jnp.float32)

    def per_graph(dst, out):
        nb = s * NPS
        for t in range(4):
            pltpu.sync_copy(zeros, acc.at[pl.ds(nb + t * ZR, ZR)])
        pltpu.sync_copy(zeros.at[pl.ds(0, NPS - 4 * ZR)],
                        acc.at[pl.ds(nb + 4 * ZR, NPS - 4 * ZR)])
        @pl.when(s == NS - 1)
        def _():
            pltpu.sync_copy(zeros.at[pl.ds(0, N_PAD - NS * NPS)],
                            acc.at[pl.ds(NS * NPS, N_PAD - NS * NPS)])
        plsc.subcore_barrier()

        def blk(j, carry):
            for b in range(NB):
                k = (j * NB + b) * NS + s
                @pl.when(j > 0)
                def _():
                    pltpu.make_async_copy(ones, acc.at[didx.at[b]],
                                          sem_s.at[b]).wait()
                pltpu.sync_copy(dst.at[pl.ds(k * CH, CH)], didx.at[b])
                pltpu.async_copy(ones, acc.at[didx.at[b]], sem_s.at[b],
                                 add=True)
            return carry

        lax.fori_loop(0, NBLK, blk, 0)
        for b in range(NB):
            pltpu.make_async_copy(ones, acc.at[didx.at[b]], sem_s.at[b]).wait()
        plsc.subcore_barrier()
        _node_chunk_copy(s, acc, out)

    @pl.when(c == 0)
    def _():
        per_graph(dst1, out1)

    @pl.when(c == 1)
    def _():
        per_graph(dst2, out2)


@jax.jit
def _sc_deg(dst1, dst2):
    return pl.kernel(
        _sc_deg_body,
        out_type=[jax.ShapeDtypeStruct((N, DEGW), jnp.float32),
                  jax.ShapeDtypeStruct((N, DEGW), jnp.float32)],
        mesh=_sc_mesh(),
        scratch_types=[
            pltpu.VMEM((NB, CH), jnp.int32),
            pltpu.VMEM((CH, DEGW), jnp.float32),
            pltpu.VMEM((ZR, DEGW), jnp.float32),
            pltpu.SemaphoreType.DMA((NB,)),
            pltpu.VMEM_SHARED((N_PAD, DEGW), jnp.float32),
        ],
    )(dst1, dst2)


def _sc_scatter_body(src1, dst1, src2, dst2, g1, g2, out1, out2,
                     sidx, didx, rows, sem_g, sem_s, acc):
    c = lax.axis_index("c")
    s = lax.axis_index("s")

    def per_graph(src, dst, g, out):
        _node_chunk_copy(s, g, acc)
        @pl.when(s == NS - 1)
        def _():
            pltpu.sync_copy(g.at[pl.ds(N, N_PAD - N)], acc.at[pl.ds(N, N_PAD - N)])
        plsc.subcore_barrier()

        def stage1(j, b):
            k = (j * NB + b) * NS + s
            pltpu.sync_copy(src.at[pl.ds(k * CH, CH)], sidx.at[b])
            pltpu.sync_copy(dst.at[pl.ds(k * CH, CH)], didx.at[b])
            pltpu.async_copy(g.at[sidx.at[b]], rows.at[b], sem_g.at[b])

        def stage2(b2):
            pltpu.make_async_copy(g.at[sidx.at[b2]], rows.at[b2],
                                  sem_g.at[b2]).wait()
            pltpu.async_copy(rows.at[b2], acc.at[didx.at[b2]], sem_s.at[b2],
                             add=True)

        def blk(j, carry):
            for b in range(NB):
                @pl.when(j > 0)
                def _():
                    pltpu.make_async_copy(rows.at[b], acc.at[didx.at[b]],
                                          sem_s.at[b]).wait()
                stage1(j, b)
                b2 = (b + NB - 2) % NB
                if b >= 2:
                    stage2(b2)
                else:
                    @pl.when(j > 0)
                    def _():
                        stage2(b2)
            return carry

        lax.fori_loop(0, NBLK, blk, 0)
        for b2 in (2, 3):
            stage2(b2)
        for b in range(NB):
            pltpu.make_async_copy(rows.at[b], acc.at[didx.at[b]],
                                  sem_s.at[b]).wait()
        plsc.subcore_barrier()
        _node_chunk_copy(s, acc, out)

    @pl.when(c == 0)
    def _():
        per_graph(src1, dst1, g1, out1)

    @pl.when(c == 1)
    def _():
        per_graph(src2, dst2, g2, out2)


@jax.jit
def _sc_scatter(src1, dst1, src2, dst2, g1, g2):
    return pl.kernel(
        _sc_scatter_body,
        out_type=[jax.ShapeDtypeStruct((N, DF), jnp.float32),
                  jax.ShapeDtypeStruct((N, DF), jnp.float32)],
        mesh=_sc_mesh(),
        scratch_types=[
            pltpu.VMEM((NB, CH), jnp.int32),
            pltpu.VMEM((NB, CH), jnp.int32),
            pltpu.VMEM((NB, CH, DF), jnp.float32),
            pltpu.SemaphoreType.DMA((NB,)),
            pltpu.SemaphoreType.DMA((NB,)),
            pltpu.VMEM_SHARED((N_PAD, DF), jnp.float32),
        ],
    )(src1, dst1, src2, dst2, g1, g2)


def _pad_edges(ei):
    pad = jnp.full((E_PAD - E,), N, jnp.int32)
    return (jnp.concatenate([ei[0], pad]), jnp.concatenate([ei[1], pad]))


def _gcn_pair(x1, ei1, x2, ei2, p):
    src1, dst1 = _pad_edges(ei1)
    src2, dst2 = _pad_edges(ei2)
    h1 = x1 @ p['conv1_W'].T
    h2 = x2 @ p['conv2_W'].T
    d1, d2 = _sc_deg(dst1, dst2)
    dinv1 = (d1[:, 0] + 1.0) ** -0.5
    dinv2 = (d2[:, 0] + 1.0) ** -0.5
    zpad = jnp.zeros((N_PAD - N, DF), jnp.float32)
    g1 = jnp.concatenate([h1 * dinv1[:, None], zpad])
    g2 = jnp.concatenate([h2 * dinv2[:, None], zpad])
    s1, s2 = _sc_scatter(src1, dst1, src2, dst2, g1, g2)
    o1 = dinv1[:, None] * s1 + p['conv1_b']
    o2 = dinv2[:, None] * s2 + p['conv2_b']
    return o1, o2


_SEQ = 4 * L
_DH = DM // NH


def _xformer_body(m1s, m1f, m2s, m2f, redW, redb,
                  inW, inb, outW, outb, ln1g, ln1b,
                  ff1W, ff1b, ff2W, ff2b, ln2g, ln2b, o_ref):
    def ln(x, g, b):
        m = jnp.mean(x, axis=-1, keepdims=True)
        d = x - m
        v = jnp.mean(d * d, axis=-1, keepdims=True)
        return d * jax.lax.rsqrt(v + EPS) * g + b

    rw = redW[...]
    rb = redb[...]
    quads = []
    flags = ((1.0, 1.0), (0.0, 1.0), (1.0, 0.0), (0.0, 0.0))
    for mref, (f1, f2) in zip((m1s, m1f, m2s, m2f), flags):
        r = mref[0] @ rw.T + rb
        c1 = jnp.full((L, 1), f1, jnp.float32)
        c2 = jnp.full((L, 1), f2, jnp.float32)
        quads.append(jnp.concatenate([r, c1, c2], axis=1))
    x = jnp.concatenate(quads, axis=0)

    for l in range(2):
        qkv = x @ inW[l].T + inb[l]
        q = qkv[:, :DM]
        k = qkv[:, DM:2 * DM]
        v = qkv[:, 2 * DM:]
        row_head = jax.lax.broadcasted_iota(jnp.int32, (NH * _SEQ, DM), 0) // _SEQ
        col_head = jax.lax.broadcasted_iota(jnp.int32, (NH * _SEQ, DM), 1) // _DH
        mask = (row_head == col_head).astype(jnp.float32)
        qs = jnp.concatenate([q, q, q, q], axis=0) * mask
        s = (qs @ k.T) * (1.0 / (_DH ** 0.5))
        s = s - jnp.max(s, axis=-1, keepdims=True)
        e = jnp.exp(s)
        p = e / jnp.sum(e, axis=-1, keepdims=True)
        ov = p @ v * mask
        att = (ov[:_SEQ] + ov[_SEQ:2 * _SEQ] + ov[2 * _SEQ:3 * _SEQ]
               + ov[3 * _SEQ:])
        a = att @ outW[l].T + outb[l]
        x = ln(x + a, ln1g[l], ln1b[l])
        f = jnp.maximum(x @ ff1W[l].T + ff1b[l], 0.0) @ ff2W[l].T + ff2b[l]
        x = ln(x + f, ln2g[l], ln2b[l])

    o_ref[...] = (jnp.sum(x, axis=0) * (1.0 / _SEQ)).reshape(1, 1, DM)


@jax.jit
def _xformer(m1s, m1f, m2s, m2f, p):
    stk = lambda k: jnp.stack([p['l0_' + k], p['l1_' + k]])
    full = lambda a: pl.BlockSpec(a.shape, lambda b: (0,) * a.ndim)
    mspec = pl.BlockSpec((1, L, DESC), lambda b: (b, 0, 0))
    ws = [stk(k) for k in ('inW', 'inb', 'outW', 'outb', 'ln1g', 'ln1b',
                           'ff1W', 'ff1b', 'ff2W', 'ff2b', 'ln2g', 'ln2b')]
    return pl.pallas_call(
        _xformer_body,
        grid=(B,),
        in_specs=[mspec] * 4 + [full(p['red_W']), full(p['red_b'])]
                 + [full(w) for w in ws],
        out_specs=pl.BlockSpec((1, 1, DM), lambda b: (b, 0, 0)),
        out_shape=jax.ShapeDtypeStruct((B, 1, DM), jnp.float32),
    )(m1s, m1f, m2s, m2f, p['red_W'], p['red_b'], *ws)


def _global_mean_pool(x, batch, num_graphs):
    s = jax.ops.segment_sum(x, batch, num_segments=num_graphs)
    c = jax.ops.segment_sum(jnp.ones((x.shape[0],), x.dtype), batch, num_segments=num_graphs)
    return s / jnp.clip(c, 1.0)[:, None]


def _layer_norm(x, g, b):
    m = x.mean(-1, keepdims=True)
    v = ((x - m) ** 2).mean(-1, keepdims=True)
    return (x - m) / jnp.sqrt(v + EPS) * g + b


def _mha(x, inW, inb, outW, outb):
    S, Bb, d = x.shape
    qkv = x @ inW.T + inb
    q, k, v = jnp.split(qkv, 3, axis=-1)
    dh = d // NH
    def rs(t):
        return t.reshape(S, Bb, NH, dh).transpose(1, 2, 0, 3)
    q = rs(q); k = rs(k); v = rs(v)
    a = jax.nn.softmax(jnp.einsum('bhsd,bhtd->bhst', q, k) / jnp.sqrt(dh), axis=-1)
    o = jnp.einsum('bhst,bhtd->bhsd', a, v)
    o = o.transpose(2, 0, 1, 3).reshape(S, Bb, d)
    return o @ outW.T + outb


def _encoder_layer(x, p, i):
    a = _mha(x, p['l%d_inW' % i], p['l%d_inb' % i], p['l%d_outW' % i], p['l%d_outb' % i])
    x = _layer_norm(x + a, p['l%d_ln1g' % i], p['l%d_ln1b' % i])
    f = jax.nn.relu(x @ p['l%d_ff1W' % i].T + p['l%d_ff1b' % i]) @ p['l%d_ff2W' % i].T + p['l%d_ff2b' % i]
    x = _layer_norm(x + f, p['l%d_ln2g' % i], p['l%d_ln2b' % i])
    return x


def _final_kernel(c_ref, w_ref, b_ref, o_ref):
    prod = c_ref[...] * w_ref[...]
    o_ref[...] = jnp.sum(prod, axis=1) + b_ref[0]


def kernel(pro1_x, pro1_edge_index, pro1_batch, pro2_x, pro2_edge_index, pro2_batch, mas1_straight, mas1_flipped, mas2_straight, mas2_flipped, params):
    p = params
    lrelu = lambda t: jax.nn.leaky_relu(t, 0.01)
    conv1, conv2 = _gcn_pair(pro1_x, pro1_edge_index, pro2_x, pro2_edge_index, p)
    x = lrelu(conv1)
    x = _global_mean_pool(x, pro1_batch, B)
    x = lrelu(x @ p['fc1_W'].T + p['fc1_b'])
    xt = lrelu(conv2)
    xt = _global_mean_pool(xt, pro2_batch, B)
    xt = lrelu(xt @ p['fc2_W'].T + p['fc2_b'])
    mas_out = _xformer(mas1_straight, mas1_flipped, mas2_straight,
                       mas2_flipped, p)[:, 0, :]
    combined = jnp.concatenate([x, xt, mas_out], axis=1)
    out = pl.pallas_call(
        _final_kernel,
        out_shape=jax.ShapeDtypeStruct((B,), jnp.float32),
    )(combined, p['final_W'], p['final_b'])
    return out[:, None]


# speedup vs baseline: 16.5230x; 1.5730x over previous
import functools

import jax
import jax.numpy as jnp
from jax import lax
from jax.experimental import pallas as pl
from jax.experimental.pallas import tpu as pltpu
from jax.experimental.pallas import tpu_sc as plsc

N = 10000; E = 320000; B = 64; L = 128; DESC = 80; DM = 32; NH = 4; FF = 128; DF = 128; OUT = 128; EPS = 1e-5

NC = 2
NS = 16
CH = 64
NB = 4
N_PAD = N + 8
NCHUNK = 5120
E_PAD = NCHUNK * CH
NIT = NCHUNK // NS
NBLK = NIT // NB
ZR = 128
NPS = 624
DEGW = 16

def _sc_mesh():
    return plsc.VectorSubcoreMesh(core_axis_name="c", subcore_axis_name="s",
                                  num_cores=NC, num_subcores=NS)


def _node_chunk_copy(s, src_ref, dst_ref):
    nb = s * NPS
    pltpu.sync_copy(src_ref.at[pl.ds(nb, NPS)], dst_ref.at[pl.ds(nb, NPS)])
    @pl.when(s == NS - 1)
    def _():
        pltpu.sync_copy(src_ref.at[pl.ds(NS * NPS, N - NS * NPS)],
                        dst_ref.at[pl.ds(NS * NPS, N - NS * NPS)])


def _sc_deg_body(dst1, dst2, out1, out2, didx, ones, zeros, sem_s, acc):
    c = lax.axis_index("c")
    s = lax.axis_index("s")
    for r in range(CH):
        ones[r] = jnp.full((DEGW,), 1.0, jnp.float32)
    for r in range(ZR):
        zeros[r] = jnp.zeros((DEGW,), jnp.float32)

    def per_graph(dst, out):
        nb = s * NPS
        for t in range(4):
            pltpu.sync_copy(zeros, acc.at[pl.ds(nb + t * ZR, ZR)])
        pltpu.sync_copy(zeros.at[pl.ds(0, NPS - 4 * ZR)],
                        acc.at[pl.ds(nb + 4 * ZR, NPS - 4 * ZR)])
        @pl.when(s == NS - 1)
        def _():
            pltpu.sync_copy(zeros.at[pl.ds(0, N_PAD - NS * NPS)],
                            acc.at[pl.ds(NS * NPS, N_PAD - NS * NPS)])
        plsc.subcore_barrier()

        def blk(j, carry):
            for b in range(NB):
                k = (j * NB + b) * NS + s
                @pl.when(j > 0)
                def _():
                    pltpu.make_async_copy(ones, acc.at[didx.at[b]],
                                          sem_s.at[b]).wait()
                pltpu.sync_copy(dst.at[pl.ds(k * CH, CH)], didx.at[b])
                pltpu.async_copy(ones, acc.at[didx.at[b]], sem_s.at[b],
                                 add=True)
            return carry

        lax.fori_loop(0, NBLK, blk, 0)
        for b in range(NB):
            pltpu.make_async_copy(ones, acc.at[didx.at[b]], sem_s.at[b]).wait()
        plsc.subcore_barrier()
        _node_chunk_copy(s, acc, out)

    @pl.when(c == 0)
    def _():
        per_graph(dst1, out1)

    @pl.when(c == 1)
    def _():
        per_graph(dst2, out2)


@jax.jit
def _sc_deg(dst1, dst2):
    return pl.kernel(
        _sc_deg_body,
        out_type=[jax.ShapeDtypeStruct((N, DEGW), jnp.float32),
                  jax.ShapeDtypeStruct((N, DEGW), jnp.float32)],
        mesh=_sc_mesh(),
        scratch_types=[
            pltpu.VMEM((NB, CH), jnp.int32),
            pltpu.VMEM((CH, DEGW), jnp.float32),
            pltpu.VMEM((ZR, DEGW), jnp.float32),
            pltpu.SemaphoreType.DMA((NB,)),
            pltpu.VMEM_SHARED((N_PAD, DEGW), jnp.float32),
        ],
    )(dst1, dst2)


def _sc_scatter_body(src1, dst1, src2, dst2, g1, g2, out1, out2,
                     sidx, didx, rows, sem_g, sem_s, acc):
    c = lax.axis_index("c")
    s = lax.axis_index("s")

    def per_graph(src, dst, g, out):
        _node_chunk_copy(s, g, acc)
        @pl.when(s == NS - 1)
        def _():
            pltpu.sync_copy(g.at[pl.ds(N, N_PAD - N)], acc.at[pl.ds(N, N_PAD - N)])
        plsc.subcore_barrier()

        def stage1(j, b):
            k = (j * NB + b) * NS + s
            pltpu.sync_copy(src.at[pl.ds(k * CH, CH)], sidx.at[b])
            pltpu.sync_copy(dst.at[pl.ds(k * CH, CH)], didx.at[b])
            pltpu.async_copy(g.at[sidx.at[b]], rows.at[b], sem_g.at[b])

        def stage2(b2):
            pltpu.make_async_copy(g.at[sidx.at[b2]], rows.at[b2],
                                  sem_g.at[b2]).wait()
            pltpu.async_copy(rows.at[b2], acc.at[didx.at[b2]], sem_s.at[b2],
                             add=True)

        def blk(j, carry):
            for b in range(NB):
                @pl.when(j > 0)
                def _():
                    pltpu.make_async_copy(rows.at[b], acc.at[didx.at[b]],
                                          sem_s.at[b]).wait()
                stage1(j, b)
                b2 = (b + NB - 2) % NB
                if b >= 2:
                    stage2(b2)
                else:
                    @pl.when(j > 0)
                    def _():
                        stage2(b2)
            return carry

        lax.fori_loop(0, NBLK, blk, 0)
        for b2 in (2, 3):
            stage2(b2)
        for b in range(NB):
            pltpu.make_async_copy(rows.at[b], acc.at[didx.at[b]],
                                  sem_s.at[b]).wait()
        plsc.subcore_barrier()
        _node_chunk_copy(s, acc, out)

    @pl.when(c == 0)
    def _():
        per_graph(src1, dst1, g1, out1)

    @pl.when(c == 1)
    def _():
        per_graph(src2, dst2, g2, out2)


@jax.jit
def _sc_scatter(src1, dst1, src2, dst2, g1, g2):
    return pl.kernel(
        _sc_scatter_body,
        out_type=[jax.ShapeDtypeStruct((N, DF), jnp.float32),
                  jax.ShapeDtypeStruct((N, DF), jnp.float32)],
        mesh=_sc_mesh(),
        scratch_types=[
            pltpu.VMEM((NB, CH), jnp.int32),
            pltpu.VMEM((NB, CH), jnp.int32),
            pltpu.VMEM((NB, CH, DF), jnp.float32),
            pltpu.SemaphoreType.DMA((NB,)),
            pltpu.SemaphoreType.DMA((NB,)),
            pltpu.VMEM_SHARED((N_PAD, DF), jnp.float32),
        ],
    )(src1, dst1, src2, dst2, g1, g2)


def _pad_edges(ei):
    pad = jnp.full((E_PAD - E,), N, jnp.int32)
    return (jnp.concatenate([ei[0], pad]), jnp.concatenate([ei[1], pad]))


RB = 1000
NRB = N // RB


def _conv_in_body(x1, W1, d1, x2, W2, d2, g1, g2):
    for x, W, d, g in ((x1, W1, d1, g1), (x2, W2, d2, g2)):
        dinv = jax.lax.rsqrt(d[...][:, :1] + 1.0)
        g[...] = (x[...] @ W[...].T) * dinv


@jax.jit
def _conv_in(x1, deg1, x2, deg2, p):
    blk = lambda c: pl.BlockSpec((RB, c), lambda i: (i, 0))
    full = lambda a: pl.BlockSpec(a.shape, lambda i: (0,) * a.ndim)
    return pl.pallas_call(
        _conv_in_body,
        grid=(NRB,),
        in_specs=[blk(DF), full(p['conv1_W']), blk(DEGW),
                  blk(DF), full(p['conv2_W']), blk(DEGW)],
        out_specs=[blk(DF), blk(DF)],
        out_shape=[jax.ShapeDtypeStruct((N, DF), jnp.float32),
                   jax.ShapeDtypeStruct((N, DF), jnp.float32)],
    )(x1, p['conv1_W'], deg1, x2, p['conv2_W'], deg2)


def _pool_body(s1, v1, b1, bat1, fW1, fb1, s2, v2, b2, bat2, fW2, fb2,
               o1, o2, acc1, cnt1, acc2, cnt2):
    i = pl.program_id(0)

    for s, v, b, bat, acc, cnt in ((s1, v1, b1, bat1, acc1, cnt1),
                                   (s2, v2, b2, bat2, acc2, cnt2)):
        y = s[...] * jax.lax.rsqrt(v[...][:, :1] + 1.0) + b[...]
        y = jnp.where(y > 0, y, 0.01 * y)
        onehot = bat[0]
        ps = onehot @ y
        pc = jnp.sum(onehot, axis=1, keepdims=True)
        @pl.when(i == 0)
        def _():
            acc[...] = ps
            cnt[...] = pc
        @pl.when(i > 0)
        def _():
            acc[...] += ps
            cnt[...] += pc

    @pl.when(i == NRB - 1)
    def _():
        for acc, cnt, fW, fb, o in ((acc1, cnt1, fW1, fb1, o1),
                                    (acc2, cnt2, fW2, fb2, o2)):
            pooled = acc[...] / jnp.maximum(cnt[...], 1.0)
            z = pooled @ fW[...].T + fb[...]
            o[...] = jnp.where(z > 0, z, 0.01 * z)


@jax.jit
def _pool_fc(s1, dinv1, bat1, s2, dinv2, bat2, p):
    blk = lambda c: pl.BlockSpec((RB, c), lambda i: (i, 0))
    bspec = pl.BlockSpec((1, B, RB), lambda i: (i, 0, 0))
    full = lambda a: pl.BlockSpec(a.shape, lambda i: (0,) * a.ndim)
    ospec = pl.BlockSpec((B, OUT), lambda i: (0, 0))
    iota_g = jnp.arange(B, dtype=jnp.int32)
    oh = lambda bat: (bat.reshape(NRB, 1, RB) == iota_g[None, :, None]
                      ).astype(jnp.float32)
    bat1_2d = oh(bat1)
    bat2_2d = oh(bat2)
    return pl.pallas_call(
        _pool_body,
        grid=(NRB,),
        in_specs=[blk(DF), blk(DEGW), full(p['conv1_b']), bspec,
                  full(p['fc1_W']), full(p['fc1_b']),
                  blk(DF), blk(DEGW), full(p['conv2_b']), bspec,
                  full(p['fc2_W']), full(p['fc2_b'])],
        out_specs=[ospec, ospec],
        out_shape=[jax.ShapeDtypeStruct((B, OUT), jnp.float32),
                   jax.ShapeDtypeStruct((B, OUT), jnp.float32)],
        scratch_shapes=[pltpu.VMEM((B, DF), jnp.float32),
                        pltpu.VMEM((B, 1), jnp.float32),
                        pltpu.VMEM((B, DF), jnp.float32),
                        pltpu.VMEM((B, 1), jnp.float32)],
    )(s1, dinv1, p['conv1_b'], bat1_2d, p['fc1_W'], p['fc1_b'],
      s2, dinv2, p['conv2_b'], bat2_2d, p['fc2_W'], p['fc2_b'])


def _gcn_pair(x1, ei1, x2, ei2, p):
    src1, dst1 = _pad_edges(ei1)
    src2, dst2 = _pad_edges(ei2)
    d1, d2 = _sc_deg(dst1, dst2)
    g1, g2 = _conv_in(x1, d1, x2, d2, p)
    zpad = jnp.zeros((N_PAD - N, DF), jnp.float32)
    s1, s2 = _sc_scatter(src1, dst1, src2, dst2,
                         jnp.concatenate([g1, zpad]),
                         jnp.concatenate([g2, zpad]))
    return s1, d1, s2, d2


_SEQ = 4 * L
_DH = DM // NH


def _xformer_body(m1s, m1f, m2s, m2f, redW, redb,
                  inW, inb, outW, outb, ln1g, ln1b,
                  ff1W, ff1b, ff2W, ff2b, ln2g, ln2b, o_ref):
    def ln(x, g, b):
        m = jnp.mean(x, axis=-1, keepdims=True)
        d = x - m
        v = jnp.mean(d * d, axis=-1, keepdims=True)
        return d * jax.lax.rsqrt(v + EPS) * g + b

    rw = redW[...]
    rb = redb[...]
    quads = []
    flags = ((1.0, 1.0), (0.0, 1.0), (1.0, 0.0), (0.0, 0.0))
    for mref, (f1, f2) in zip((m1s, m1f, m2s, m2f), flags):
        r = mref[0] @ rw.T + rb
        c1 = jnp.full((L, 1), f1, jnp.float32)
        c2 = jnp.full((L, 1), f2, jnp.float32)
        quads.append(jnp.concatenate([r, c1, c2], axis=1))
    x = jnp.concatenate(quads, axis=0)

    for l in range(2):
        qkv = x @ inW[l].T + inb[l]
        q = qkv[:, :DM]
        k = qkv[:, DM:2 * DM]
        v = qkv[:, 2 * DM:]
        row_head = jax.lax.broadcasted_iota(jnp.int32, (NH * _SEQ, DM), 0) // _SEQ
        col_head = jax.lax.broadcasted_iota(jnp.int32, (NH * _SEQ, DM), 1) // _DH
        mask = (row_head == col_head).astype(jnp.float32)
        qs = jnp.concatenate([q, q, q, q], axis=0) * mask
        s = (qs @ k.T) * (1.0 / (_DH ** 0.5))
        s = s - jnp.max(s, axis=-1, keepdims=True)
        e = jnp.exp(s)
        p = e / jnp.sum(e, axis=-1, keepdims=True)
        ov = p @ v * mask
        att = (ov[:_SEQ] + ov[_SEQ:2 * _SEQ] + ov[2 * _SEQ:3 * _SEQ]
               + ov[3 * _SEQ:])
        a = att @ outW[l].T + outb[l]
        x = ln(x + a, ln1g[l], ln1b[l])
        f = jnp.maximum(x @ ff1W[l].T + ff1b[l], 0.0) @ ff2W[l].T + ff2b[l]
        x = ln(x + f, ln2g[l], ln2b[l])

    o_ref[...] = (jnp.sum(x, axis=0) * (1.0 / _SEQ)).reshape(1, 1, DM)


@jax.jit
def _xformer(m1s, m1f, m2s, m2f, p):
    stk = lambda k: jnp.stack([p['l0_' + k], p['l1_' + k]])
    full = lambda a: pl.BlockSpec(a.shape, lambda b: (0,) * a.ndim)
    mspec = pl.BlockSpec((1, L, DESC), lambda b: (b, 0, 0))
    ws = [stk(k) for k in ('inW', 'inb', 'outW', 'outb', 'ln1g', 'ln1b',
                           'ff1W', 'ff1b', 'ff2W', 'ff2b', 'ln2g', 'ln2b')]
    return pl.pallas_call(
        _xformer_body,
        grid=(B,),
        in_specs=[mspec] * 4 + [full(p['red_W']), full(p['red_b'])]
                 + [full(w) for w in ws],
        out_specs=pl.BlockSpec((1, 1, DM), lambda b: (b, 0, 0)),
        out_shape=jax.ShapeDtypeStruct((B, 1, DM), jnp.float32),
    )(m1s, m1f, m2s, m2f, p['red_W'], p['red_b'], *ws)


def _global_mean_pool(x, batch, num_graphs):
    s = jax.ops.segment_sum(x, batch, num_segments=num_graphs)
    c = jax.ops.segment_sum(jnp.ones((x.shape[0],), x.dtype), batch, num_segments=num_graphs)
    return s / jnp.clip(c, 1.0)[:, None]


def _layer_norm(x, g, b):
    m = x.mean(-1, keepdims=True)
    v = ((x - m) ** 2).mean(-1, keepdims=True)
    return (x - m) / jnp.sqrt(v + EPS) * g + b


def _mha(x, inW, inb, outW, outb):
    S, Bb, d = x.shape
    qkv = x @ inW.T + inb
    q, k, v = jnp.split(qkv, 3, axis=-1)
    dh = d // NH
    def rs(t):
        return t.reshape(S, Bb, NH, dh).transpose(1, 2, 0, 3)
    q = rs(q); k = rs(k); v = rs(v)
    a = jax.nn.softmax(jnp.einsum('bhsd,bhtd->bhst', q, k) / jnp.sqrt(dh), axis=-1)
    o = jnp.einsum('bhst,bhtd->bhsd', a, v)
    o = o.transpose(2, 0, 1, 3).reshape(S, Bb, d)
    return o @ outW.T + outb


def _encoder_layer(x, p, i):
    a = _mha(x, p['l%d_inW' % i], p['l%d_inb' % i], p['l%d_outW' % i], p['l%d_outb' % i])
    x = _layer_norm(x + a, p['l%d_ln1g' % i], p['l%d_ln1b' % i])
    f = jax.nn.relu(x @ p['l%d_ff1W' % i].T + p['l%d_ff1b' % i]) @ p['l%d_ff2W' % i].T + p['l%d_ff2b' % i]
    x = _layer_norm(x + f, p['l%d_ln2g' % i], p['l%d_ln2b' % i])
    return x


def _final_kernel(c_ref, w_ref, b_ref, o_ref):
    prod = c_ref[...] * w_ref[...]
    o_ref[...] = jnp.sum(prod, axis=1) + b_ref[0]


def kernel(pro1_x, pro1_edge_index, pro1_batch, pro2_x, pro2_edge_index, pro2_batch, mas1_straight, mas1_flipped, mas2_straight, mas2_flipped, params):
    p = params
    s1, dinv1, s2, dinv2 = _gcn_pair(pro1_x, pro1_edge_index,
                                     pro2_x, pro2_edge_index, p)
    x, xt = _pool_fc(s1, dinv1, pro1_batch, s2, dinv2, pro2_batch, p)
    mas_out = _xformer(mas1_straight, mas1_flipped, mas2_straight,
                       mas2_flipped, p)[:, 0, :]
    combined = jnp.concatenate([x, xt, mas_out], axis=1)
    out = pl.pallas_call(
        _final_kernel,
        out_shape=jax.ShapeDtypeStruct((B,), jnp.float32),
    )(combined, p['final_W'], p['final_b'])
    return out[:, None]


# speedup vs baseline: 16.8637x; 1.0206x over previous
import functools

import jax
import jax.numpy as jnp
from jax import lax
from jax.experimental import pallas as pl
from jax.experimental.pallas import tpu as pltpu
from jax.experimental.pallas import tpu_sc as plsc

N = 10000; E = 320000; B = 64; L = 128; DESC = 80; DM = 32; NH = 4; FF = 128; DF = 128; OUT = 128; EPS = 1e-5

NC = 2
NS = 16
CH = 128
NB = 4
NBS = 2
N_PAD = N + 8
NCHUNK = 2560
E_PAD = NCHUNK * CH
NIT = NCHUNK // NS
SB = 16
NSB = NIT // SB
ZR = 128
NPS = 624
DEGW = 16

def _sc_mesh():
    return plsc.VectorSubcoreMesh(core_axis_name="c", subcore_axis_name="s",
                                  num_cores=NC, num_subcores=NS)


def _node_chunk_copy(s, src_ref, dst_ref):
    nb = s * NPS
    pltpu.sync_copy(src_ref.at[pl.ds(nb, NPS)], dst_ref.at[pl.ds(nb, NPS)])
    @pl.when(s == NS - 1)
    def _():
        pltpu.sync_copy(src_ref.at[pl.ds(NS * NPS, N - NS * NPS)],
                        dst_ref.at[pl.ds(NS * NPS, N - NS * NPS)])


def _sc_deg_body(dst1, dst2, out1, out2, didx, ones, zeros, sem_s, acc):
    c = lax.axis_index("c")
    s = lax.axis_index("s")
    for r in range(CH):
        ones[r] = jnp.full((DEGW,), 1.0, jnp.float32)
    for r in range(ZR):
        zeros[r] = jnp.zeros((DEGW,), jnp.float32)

    def per_graph(dst, out):
        nb = s * NPS
        for t in range(4):
            pltpu.sync_copy(zeros, acc.at[pl.ds(nb + t * ZR, ZR)])
        pltpu.sync_copy(zeros.at[pl.ds(0, NPS - 4 * ZR)],
                        acc.at[pl.ds(nb + 4 * ZR, NPS - 4 * ZR)])
        @pl.when(s == NS - 1)
        def _():
            pltpu.sync_copy(zeros.at[pl.ds(0, N_PAD - NS * NPS)],
                            acc.at[pl.ds(NS * NPS, N_PAD - NS * NPS)])
        plsc.subcore_barrier()

        def blk(jb, carry):
            pltpu.sync_copy(dst.at[pl.ds(s * NIT + jb * SB, SB)], didx)
            for t in range(SB):
                b = t % NB
                if t >= NB:
                    pltpu.make_async_copy(ones, acc.at[didx.at[b]],
                                          sem_s.at[b]).wait()
                pltpu.async_copy(ones, acc.at[didx.at[t]], sem_s.at[b],
                                 add=True)
            for b in range(NB):
                pltpu.make_async_copy(ones, acc.at[didx.at[b]],
                                      sem_s.at[b]).wait()
            return carry

        lax.fori_loop(0, NSB, blk, 0)
        plsc.subcore_barrier()
        _node_chunk_copy(s, acc, out)

    @pl.when(c == 0)
    def _():
        per_graph(dst1, out1)

    @pl.when(c == 1)
    def _():
        per_graph(dst2, out2)


@jax.jit
def _sc_deg(dst1, dst2):
    return pl.kernel(
        _sc_deg_body,
        out_type=[jax.ShapeDtypeStruct((N, DEGW), jnp.float32),
                  jax.ShapeDtypeStruct((N, DEGW), jnp.float32)],
        mesh=_sc_mesh(),
        scratch_types=[
            pltpu.VMEM((SB, CH), jnp.int32),
            pltpu.VMEM((CH, DEGW), jnp.float32),
            pltpu.VMEM((ZR, DEGW), jnp.float32),
            pltpu.SemaphoreType.DMA((NB,)),
            pltpu.VMEM_SHARED((N_PAD, DEGW), jnp.float32),
        ],
    )(dst1, dst2)


def _sc_scatter_body(src1, dst1, src2, dst2, g1, g2, out1, out2,
                     sidx, didx, rows, sem_g, sem_s, acc):
    c = lax.axis_index("c")
    s = lax.axis_index("s")

    def per_graph(src, dst, g, out):
        _node_chunk_copy(s, g, acc)
        @pl.when(s == NS - 1)
        def _():
            pltpu.sync_copy(g.at[pl.ds(N, N_PAD - N)], acc.at[pl.ds(N, N_PAD - N)])
        plsc.subcore_barrier()

        def scat(t):
            b = t % NBS
            pltpu.make_async_copy(g.at[sidx.at[t]], rows.at[b],
                                  sem_g.at[b]).wait()
            pltpu.async_copy(rows.at[b], acc.at[didx.at[t]], sem_s.at[b],
                             add=True)

        def blk(jb, carry):
            pltpu.sync_copy(src.at[pl.ds(s * NIT + jb * SB, SB)], sidx)
            pltpu.sync_copy(dst.at[pl.ds(s * NIT + jb * SB, SB)], didx)
            for t in range(SB):
                b = t % NBS
                if t >= NBS:
                    pltpu.make_async_copy(rows.at[b], acc.at[didx.at[b]],
                                          sem_s.at[b]).wait()
                pltpu.async_copy(g.at[sidx.at[t]], rows.at[b], sem_g.at[b])
                if t >= 1:
                    scat(t - 1)
            scat(SB - 1)
            for b in range(NBS):
                pltpu.make_async_copy(rows.at[b], acc.at[didx.at[b]],
                                      sem_s.at[b]).wait()
            return carry

        lax.fori_loop(0, NSB, blk, 0)
        plsc.subcore_barrier()
        _node_chunk_copy(s, acc, out)

    @pl.when(c == 0)
    def _():
        per_graph(src1, dst1, g1, out1)

    @pl.when(c == 1)
    def _():
        per_graph(src2, dst2, g2, out2)


@jax.jit
def _sc_scatter(src1, dst1, src2, dst2, g1, g2):
    return pl.kernel(
        _sc_scatter_body,
        out_type=[jax.ShapeDtypeStruct((N, DF), jnp.float32),
                  jax.ShapeDtypeStruct((N, DF), jnp.float32)],
        mesh=_sc_mesh(),
        scratch_types=[
            pltpu.VMEM((SB, CH), jnp.int32),
            pltpu.VMEM((SB, CH), jnp.int32),
            pltpu.VMEM((NBS, CH, DF), jnp.float32),
            pltpu.SemaphoreType.DMA((NBS,)),
            pltpu.SemaphoreType.DMA((NBS,)),
            pltpu.VMEM_SHARED((N_PAD, DF), jnp.float32),
        ],
    )(src1, dst1, src2, dst2, g1, g2)


def _pad_edges(ei):
    pad = jnp.full((E_PAD - E,), N, jnp.int32)
    return (jnp.concatenate([ei[0], pad]).reshape(NCHUNK, CH),
            jnp.concatenate([ei[1], pad]).reshape(NCHUNK, CH))


RB = 1000
NRB = N // RB


def _conv_in_body(x1, W1, d1, x2, W2, d2, g1, g2):
    for x, W, d, g in ((x1, W1, d1, g1), (x2, W2, d2, g2)):
        dinv = jax.lax.rsqrt(d[...][:, :1] + 1.0)
        g[...] = (x[...] @ W[...].T) * dinv


@jax.jit
def _conv_in(x1, deg1, x2, deg2, p):
    blk = lambda c: pl.BlockSpec((RB, c), lambda i: (i, 0))
    full = lambda a: pl.BlockSpec(a.shape, lambda i: (0,) * a.ndim)
    return pl.pallas_call(
        _conv_in_body,
        grid=(NRB,),
        in_specs=[blk(DF), full(p['conv1_W']), blk(DEGW),
                  blk(DF), full(p['conv2_W']), blk(DEGW)],
        out_specs=[blk(DF), blk(DF)],
        out_shape=[jax.ShapeDtypeStruct((N, DF), jnp.float32),
                   jax.ShapeDtypeStruct((N, DF), jnp.float32)],
    )(x1, p['conv1_W'], deg1, x2, p['conv2_W'], deg2)


def _pool_body(s1, v1, b1, bat1, fW1, fb1, s2, v2, b2, bat2, fW2, fb2,
               o1, o2, acc1, cnt1, acc2, cnt2):
    i = pl.program_id(0)

    for s, v, b, bat, acc, cnt in ((s1, v1, b1, bat1, acc1, cnt1),
                                   (s2, v2, b2, bat2, acc2, cnt2)):
        y = s[...] * jax.lax.rsqrt(v[...][:, :1] + 1.0) + b[...]
        y = jnp.where(y > 0, y, 0.01 * y)
        onehot = bat[0]
        ps = onehot @ y
        pc = jnp.sum(onehot, axis=1, keepdims=True)
        @pl.when(i == 0)
        def _():
            acc[...] = ps
            cnt[...] = pc
        @pl.when(i > 0)
        def _():
            acc[...] += ps
            cnt[...] += pc

    @pl.when(i == NRB - 1)
    def _():
        for acc, cnt, fW, fb, o in ((acc1, cnt1, fW1, fb1, o1),
                                    (acc2, cnt2, fW2, fb2, o2)):
            pooled = acc[...] / jnp.maximum(cnt[...], 1.0)
            z = pooled @ fW[...].T + fb[...]
            o[...] = jnp.where(z > 0, z, 0.01 * z)


@jax.jit
def _pool_fc(s1, dinv1, bat1, s2, dinv2, bat2, p):
    blk = lambda c: pl.BlockSpec((RB, c), lambda i: (i, 0))
    bspec = pl.BlockSpec((1, B, RB), lambda i: (i, 0, 0))
    full = lambda a: pl.BlockSpec(a.shape, lambda i: (0,) * a.ndim)
    ospec = pl.BlockSpec((B, OUT), lambda i: (0, 0))
    iota_g = jnp.arange(B, dtype=jnp.int32)
    oh = lambda bat: (bat.reshape(NRB, 1, RB) == iota_g[None, :, None]
                      ).astype(jnp.float32)
    bat1_2d = oh(bat1)
    bat2_2d = oh(bat2)
    return pl.pallas_call(
        _pool_body,
        grid=(NRB,),
        in_specs=[blk(DF), blk(DEGW), full(p['conv1_b']), bspec,
                  full(p['fc1_W']), full(p['fc1_b']),
                  blk(DF), blk(DEGW), full(p['conv2_b']), bspec,
                  full(p['fc2_W']), full(p['fc2_b'])],
        out_specs=[ospec, ospec],
        out_shape=[jax.ShapeDtypeStruct((B, OUT), jnp.float32),
                   jax.ShapeDtypeStruct((B, OUT), jnp.float32)],
        scratch_shapes=[pltpu.VMEM((B, DF), jnp.float32),
                        pltpu.VMEM((B, 1), jnp.float32),
                        pltpu.VMEM((B, DF), jnp.float32),
                        pltpu.VMEM((B, 1), jnp.float32)],
    )(s1, dinv1, p['conv1_b'], bat1_2d, p['fc1_W'], p['fc1_b'],
      s2, dinv2, p['conv2_b'], bat2_2d, p['fc2_W'], p['fc2_b'])


def _gcn_pair(x1, ei1, x2, ei2, p):
    src1, dst1 = _pad_edges(ei1)
    src2, dst2 = _pad_edges(ei2)
    d1, d2 = _sc_deg(dst1, dst2)
    g1, g2 = _conv_in(x1, d1, x2, d2, p)
    zpad = jnp.zeros((N_PAD - N, DF), jnp.float32)
    s1, s2 = _sc_scatter(src1, dst1, src2, dst2,
                         jnp.concatenate([g1, zpad]),
                         jnp.concatenate([g2, zpad]))
    return s1, d1, s2, d2


_SEQ = 4 * L
_DH = DM // NH


def _xformer_body(m1s, m1f, m2s, m2f, redW, redb,
                  inW, inb, outW, outb, ln1g, ln1b,
                  ff1W, ff1b, ff2W, ff2b, ln2g, ln2b, o_ref):
    def ln(x, g, b):
        m = jnp.mean(x, axis=-1, keepdims=True)
        d = x - m
        v = jnp.mean(d * d, axis=-1, keepdims=True)
        return d * jax.lax.rsqrt(v + EPS) * g + b

    rw = redW[...]
    rb = redb[...]
    quads = []
    flags = ((1.0, 1.0), (0.0, 1.0), (1.0, 0.0), (0.0, 0.0))
    for mref, (f1, f2) in zip((m1s, m1f, m2s, m2f), flags):
        r = mref[0] @ rw.T + rb
        c1 = jnp.full((L, 1), f1, jnp.float32)
        c2 = jnp.full((L, 1), f2, jnp.float32)
        quads.append(jnp.concatenate([r, c1, c2], axis=1))
    x = jnp.concatenate(quads, axis=0)

    for l in range(2):
        qkv = x @ inW[l].T + inb[l]
        q = qkv[:, :DM]
        k = qkv[:, DM:2 * DM]
        v = qkv[:, 2 * DM:]
        row_head = jax.lax.broadcasted_iota(jnp.int32, (NH * _SEQ, DM), 0) // _SEQ
        col_head = jax.lax.broadcasted_iota(jnp.int32, (NH * _SEQ, DM), 1) // _DH
        mask = (row_head == col_head).astype(jnp.float32)
        qs = jnp.concatenate([q, q, q, q], axis=0) * mask
        s = (qs @ k.T) * (1.0 / (_DH ** 0.5))
        s = s - jnp.max(s, axis=-1, keepdims=True)
        e = jnp.exp(s)
        p = e / jnp.sum(e, axis=-1, keepdims=True)
        ov = p @ v * mask
        att = (ov[:_SEQ] + ov[_SEQ:2 * _SEQ] + ov[2 * _SEQ:3 * _SEQ]
               + ov[3 * _SEQ:])
        a = att @ outW[l].T + outb[l]
        x = ln(x + a, ln1g[l], ln1b[l])
        f = jnp.maximum(x @ ff1W[l].T + ff1b[l], 0.0) @ ff2W[l].T + ff2b[l]
        x = ln(x + f, ln2g[l], ln2b[l])

    o_ref[...] = (jnp.sum(x, axis=0) * (1.0 / _SEQ)).reshape(1, 1, DM)


@jax.jit
def _xformer(m1s, m1f, m2s, m2f, p):
    stk = lambda k: jnp.stack([p['l0_' + k], p['l1_' + k]])
    full = lambda a: pl.BlockSpec(a.shape, lambda b: (0,) * a.ndim)
    mspec = pl.BlockSpec((1, L, DESC), lambda b: (b, 0, 0))
    ws = [stk(k) for k in ('inW', 'inb', 'outW', 'outb', 'ln1g', 'ln1b',
                           'ff1W', 'ff1b', 'ff2W', 'ff2b', 'ln2g', 'ln2b')]
    return pl.pallas_call(
        _xformer_body,
        grid=(B,),
        in_specs=[mspec] * 4 + [full(p['red_W']), full(p['red_b'])]
                 + [full(w) for w in ws],
        out_specs=pl.BlockSpec((1, 1, DM), lambda b: (b, 0, 0)),
        out_shape=jax.ShapeDtypeStruct((B, 1, DM), jnp.float32),
    )(m1s, m1f, m2s, m2f, p['red_W'], p['red_b'], *ws)


def _global_mean_pool(x, batch, num_graphs):
    s = jax.ops.segment_sum(x, batch, num_segments=num_graphs)
    c = jax.ops.segment_sum(jnp.ones((x.shape[0],), x.dtype), batch, num_segments=num_graphs)
    return s / jnp.clip(c, 1.0)[:, None]


def _layer_norm(x, g, b):
    m = x.mean(-1, keepdims=True)
    v = ((x - m) ** 2).mean(-1, keepdims=True)
    return (x - m) / jnp.sqrt(v + EPS) * g + b


def _mha(x, inW, inb, outW, outb):
    S, Bb, d = x.shape
    qkv = x @ inW.T + inb
    q, k, v = jnp.split(qkv, 3, axis=-1)
    dh = d // NH
    def rs(t):
        return t.reshape(S, Bb, NH, dh).transpose(1, 2, 0, 3)
    q = rs(q); k = rs(k); v = rs(v)
    a = jax.nn.softmax(jnp.einsum('bhsd,bhtd->bhst', q, k) / jnp.sqrt(dh), axis=-1)
    o = jnp.einsum('bhst,bhtd->bhsd', a, v)
    o = o.transpose(2, 0, 1, 3).reshape(S, Bb, d)
    return o @ outW.T + outb


def _encoder_layer(x, p, i):
    a = _mha(x, p['l%d_inW' % i], p['l%d_inb' % i], p['l%d_outW' % i], p['l%d_outb' % i])
    x = _layer_norm(x + a, p['l%d_ln1g' % i], p['l%d_ln1b' % i])
    f = jax.nn.relu(x @ p['l%d_ff1W' % i].T + p['l%d_ff1b' % i]) @ p['l%d_ff2W' % i].T + p['l%d_ff2b' % i]
    x = _layer_norm(x + f, p['l%d_ln2g' % i], p['l%d_ln2b' % i])
    return x


def _final_kernel(c_ref, w_ref, b_ref, o_ref):
    prod = c_ref[...] * w_ref[...]
    o_ref[...] = jnp.sum(prod, axis=1) + b_ref[0]


def kernel(pro1_x, pro1_edge_index, pro1_batch, pro2_x, pro2_edge_index, pro2_batch, mas1_straight, mas1_flipped, mas2_straight, mas2_flipped, params):
    p = params
    s1, dinv1, s2, dinv2 = _gcn_pair(pro1_x, pro1_edge_index,
                                     pro2_x, pro2_edge_index, p)
    x, xt = _pool_fc(s1, dinv1, pro1_batch, s2, dinv2, pro2_batch, p)
    mas_out = _xformer(mas1_straight, mas1_flipped, mas2_straight,
                       mas2_flipped, p)[:, 0, :]
    combined = jnp.concatenate([x, xt, mas_out], axis=1)
    out = pl.pallas_call(
        _final_kernel,
        out_shape=jax.ShapeDtypeStruct((B,), jnp.float32),
    )(combined, p['final_W'], p['final_b'])
    return out[:, None]
